# Initial kernel scaffold; baseline (speedup 1.0000x reference)
import jax
import jax.numpy as jnp
from jax.experimental import pallas as pl


def kernel(x, edge_index, Wl0, bl0, Wr0, Wl1, bl1, Wr1, Wp1, bp1, Wp2, bp2):
    raise NotImplementedError("write your pallas kernel here")


# speedup vs baseline: 4.7601x; 4.7601x over previous
import jax
import jax.numpy as jnp
from jax import lax
from jax.experimental import pallas as pl
from jax.experimental.pallas import tpu as pltpu
from jax.experimental.pallas import tpu_sc as plsc

_NC = 2
_NS = 16
_K = 80
_Z = 48


def _mesh():
    return plsc.VectorSubcoreMesh(core_axis_name="c", subcore_axis_name="s",
                                  num_cores=_NC)


def _agg_sc(x, src, dst, zf):
    N, D = x.shape
    E = src.shape[0]
    epw = E // (_NC * _NS)
    ch = epw // _K
    rpt = (N // _NS) & ~7
    tail = N - rpt * _NS
    zloops = rpt // _Z

    def body(x_hbm, src_hbm, dst_hbm, zf_hbm, agg_out, sidx, didx, rows,
             agg_sp, sem):
        c = lax.axis_index("c")
        s = lax.axis_index("s")
        w = c * _NS + s
        rbase = s * rpt
        pltpu.sync_copy(zf_hbm, rows.at[pl.ds(0, _Z)])

        def zero(j, carry):
            pltpu.sync_copy(rows.at[pl.ds(0, _Z)],
                            agg_sp.at[pl.ds(rbase + j * _Z, _Z)])
            return carry

        lax.fori_loop(0, zloops, zero, 0)
        if tail:
            @pl.when(s == _NS - 1)
            def _():
                pltpu.sync_copy(rows.at[pl.ds(0, tail)],
                                agg_sp.at[pl.ds(rpt * _NS, tail)])
        plsc.subcore_barrier()

        ebase = w * epw

        def chunk(i, carry):
            off = ebase + i * _K
            pltpu.sync_copy(src_hbm.at[pl.ds(off, _K)], sidx)
            pltpu.sync_copy(dst_hbm.at[pl.ds(off, _K)], didx)
            pltpu.async_copy(x_hbm.at[sidx], rows, sem).wait()
            pltpu.sync_copy(rows, agg_sp.at[didx], add=True)
            return carry

        lax.fori_loop(0, ch, chunk, 0)
        plsc.subcore_barrier()

        def copy_out(j, carry):
            pltpu.sync_copy(agg_sp.at[pl.ds(rbase + j * _Z, _Z)],
                            rows.at[pl.ds(0, _Z)])
            pltpu.sync_copy(rows.at[pl.ds(0, _Z)],
                            agg_out.at[c, pl.ds(rbase + j * _Z, _Z)])
            return carry

        lax.fori_loop(0, zloops, copy_out, 0)
        if tail:
            @pl.when(s == _NS - 1)
            def _():
                pltpu.sync_copy(agg_sp.at[pl.ds(rpt * _NS, tail)],
                                rows.at[pl.ds(0, tail)])
                pltpu.sync_copy(rows.at[pl.ds(0, tail)],
                                agg_out.at[c, pl.ds(rpt * _NS, tail)])

    fn = pl.kernel(
        body,
        out_type=(jax.ShapeDtypeStruct((_NC, N, D), jnp.float32),),
        mesh=_mesh(),
        scratch_types=[
            pltpu.VMEM((_K,), jnp.int32),
            pltpu.VMEM((_K,), jnp.int32),
            pltpu.VMEM((_K, D), jnp.float32),
            pltpu.VMEM_SHARED((N, D), jnp.float32),
            pltpu.SemaphoreType.DMA,
        ],
    )
    (o,) = fn(x, src, dst, zf)
    return o


def _cnt_sc(dst, zc, ones, N, D):
    E = dst.shape[0]
    epw = E // (_NC * _NS)
    ch = epw // _K
    rpt = (N // _NS) & ~7
    tail = N - rpt * _NS
    zloops = rpt // _Z

    def body(dst_hbm, zc_hbm, ones_hbm, cnt_out, didx, ones_v, cbuf,
             cnt_sp, sem):
        c = lax.axis_index("c")
        s = lax.axis_index("s")
        w = c * _NS + s
        rbase = s * rpt
        pltpu.sync_copy(zc_hbm, cbuf)

        def zero(j, carry):
            pltpu.sync_copy(cbuf.at[pl.ds(0, _Z)],
                            cnt_sp.at[pl.ds(rbase + j * _Z, _Z)])
            return carry

        lax.fori_loop(0, zloops, zero, 0)
        if tail:
            @pl.when(s == _NS - 1)
            def _():
                pltpu.sync_copy(cbuf.at[pl.ds(0, tail)],
                                cnt_sp.at[pl.ds(rpt * _NS, tail)])
        pltpu.sync_copy(ones_hbm, ones_v)
        plsc.subcore_barrier()

        ebase = w * epw

        def chunk(i, carry):
            off = ebase + i * _K
            pltpu.sync_copy(dst_hbm.at[pl.ds(off, _K)], didx)
            pltpu.sync_copy(ones_v, cnt_sp.at[didx], add=True)
            return carry

        lax.fori_loop(0, ch, chunk, 0)
        plsc.subcore_barrier()

        def copy_out(j, carry):
            pltpu.sync_copy(cnt_sp.at[pl.ds(rbase + j * _Z, _Z)],
                            cbuf.at[pl.ds(0, _Z)])
            pltpu.sync_copy(cbuf.at[pl.ds(0, _Z)],
                            cnt_out.at[c, pl.ds(rbase + j * _Z, _Z)])
            return carry

        lax.fori_loop(0, zloops, copy_out, 0)
        if tail:
            @pl.when(s == _NS - 1)
            def _():
                pltpu.sync_copy(cnt_sp.at[pl.ds(rpt * _NS, tail)],
                                cbuf.at[pl.ds(0, tail)])
                pltpu.sync_copy(cbuf.at[pl.ds(0, tail)],
                                cnt_out.at[c, pl.ds(rpt * _NS, tail)])

    fn = pl.kernel(
        body,
        out_type=(jax.ShapeDtypeStruct((_NC, N, D), jnp.float32),),
        mesh=_mesh(),
        scratch_types=[
            pltpu.VMEM((_K,), jnp.int32),
            pltpu.VMEM((_K, D), jnp.float32),
            pltpu.VMEM((_Z, D), jnp.float32),
            pltpu.VMEM_SHARED((N, D), jnp.float32),
            pltpu.SemaphoreType.DMA,
        ],
    )
    (o,) = fn(dst, zc, ones)
    return o


def _dotT(a, w):
    return lax.dot_general(a, w, (((1,), (1,)), ((), ())),
                           preferred_element_type=jnp.float32)


def _tc_layer(x, agg, cnt, Wl, bl, Wr):
    N, D = x.shape
    R = 1000

    def body(x_ref, a0, a1, c0, c1, wl, blr, wr, o_ref):
        n = jnp.maximum(c0[0][:, 0:1] + c1[0][:, 0:1], 1.0)
        mean = (a0[0] + a1[0]) / n
        h = _dotT(mean, wl[...]) + blr[...] + _dotT(x_ref[...], wr[...])
        o_ref[...] = jnp.maximum(h, 0.0)

    return pl.pallas_call(
        body,
        grid=(N // R,),
        in_specs=[
            pl.BlockSpec((R, D), lambda i: (i, 0)),
            pl.BlockSpec((1, R, D), lambda i: (0, i, 0)),
            pl.BlockSpec((1, R, D), lambda i: (1, i, 0)),
            pl.BlockSpec((1, R, D), lambda i: (0, i, 0)),
            pl.BlockSpec((1, R, D), lambda i: (1, i, 0)),
            pl.BlockSpec((D, D), lambda i: (0, 0)),
            pl.BlockSpec((1, D), lambda i: (0, 0)),
            pl.BlockSpec((D, D), lambda i: (0, 0)),
        ],
        out_specs=pl.BlockSpec((R, D), lambda i: (i, 0)),
        out_shape=jax.ShapeDtypeStruct((N, D), jnp.float32),
    )(x, agg, agg, cnt, cnt, Wl, bl.reshape(1, D), Wr)


def _tc_final(h, agg, cnt, Wl, bl, Wr, Wp1, bp1, Wp2, bp2):
    N, D = h.shape
    R = 1000

    def body(h_ref, a0, a1, c0, c1, wl, blr, wr, wp1, bp1r, wp2, bp2r,
             o_ref):
        n = jnp.maximum(c0[0][:, 0:1] + c1[0][:, 0:1], 1.0)
        mean = (a0[0] + a1[0]) / n
        g = _dotT(mean, wl[...]) + blr[...] + _dotT(h_ref[...], wr[...])
        g = jnp.maximum(g, 0.0)
        g = _dotT(g, wp1[...]) + bp1r[...]
        o_ref[...] = _dotT(g, wp2[...]) + bp2r[...]

    return pl.pallas_call(
        body,
        grid=(N // R,),
        in_specs=[
            pl.BlockSpec((R, D), lambda i: (i, 0)),
            pl.BlockSpec((1, R, D), lambda i: (0, i, 0)),
            pl.BlockSpec((1, R, D), lambda i: (1, i, 0)),
            pl.BlockSpec((1, R, D), lambda i: (0, i, 0)),
            pl.BlockSpec((1, R, D), lambda i: (1, i, 0)),
            pl.BlockSpec((D, D), lambda i: (0, 0)),
            pl.BlockSpec((1, D), lambda i: (0, 0)),
            pl.BlockSpec((D, D), lambda i: (0, 0)),
            pl.BlockSpec((D, D), lambda i: (0, 0)),
            pl.BlockSpec((1, D), lambda i: (0, 0)),
            pl.BlockSpec((D, D), lambda i: (0, 0)),
            pl.BlockSpec((1, D), lambda i: (0, 0)),
        ],
        out_specs=pl.BlockSpec((R, D), lambda i: (i, 0)),
        out_shape=jax.ShapeDtypeStruct((N, D), jnp.float32),
    )(h, agg, agg, cnt, cnt, Wl, bl.reshape(1, D), Wr,
      Wp1, bp1.reshape(1, D), Wp2, bp2.reshape(1, D))


def kernel(x, edge_index, Wl0, bl0, Wr0, Wl1, bl1, Wr1, Wp1, bp1, Wp2, bp2):
    N, D = x.shape
    src = edge_index[0]
    dst = edge_index[1]
    zf = jnp.zeros((_Z, D), jnp.float32)
    ones = jnp.ones((_K, D), jnp.float32)

    cnt = _cnt_sc(dst, zf, ones, N, D)
    agg0 = _agg_sc(x, src, dst, zf)
    h = _tc_layer(x, agg0, cnt, Wl0, bl0, Wr0)
    agg1 = _agg_sc(h, src, dst, zf)
    return _tc_final(h, agg1, cnt, Wl1, bl1, Wr1, Wp1, bp1, Wp2, bp2)
